# baseline (speedup 1.0000x reference)
import jax
import jax.numpy as jnp
from jax.experimental import pallas as pl


def kernel(x, tables):
    n, f = x.shape
    return jnp.reshape(x, (n * f // 128, 128))


# speedup vs baseline: 3.3521x; 3.3521x over previous
import functools

import jax
import jax.numpy as jnp
from jax.experimental import pallas as pl
from jax.experimental.pallas import tpu as pltpu


def _round_up(a: int, m: int) -> int:
    return (a + m - 1) // m * m


def _encode_kernel(xt_ref, tab_ref, out_ref, *, num_features, vocab):
    tile_n = xt_ref.shape[1]
    total = num_features * vocab

    xt = xt_ref[...]
    selT = jnp.concatenate(
        [
            jnp.broadcast_to(xt[i : i + 1, :], (vocab, tile_n))
            for i in range(num_features)
        ],
        axis=0,
    )

    subT = jax.lax.broadcasted_iota(jnp.int32, (total, tile_n), 0) % vocab

    hotT = (selT == subT).astype(jnp.bfloat16)
    out_ref[...] = jax.lax.dot_general(
        hotT,
        tab_ref[...],
        (((0,), (0,)), ((), ())),
        preferred_element_type=jnp.float32,
    )


def kernel(x, tables):
    if x.ndim == 1:
        x = x[:, None]
    n, f = x.shape
    fe, v, h = tables.shape
    assert f == fe, "number of index columns must match number of tables"

    tab2d = tables.reshape(fe * v, h).astype(jnp.bfloat16)
    xt = x.astype(jnp.int32).T

    tile = max(128, min(2048, _round_up(n, 128)))
    n_pad = _round_up(n, tile)
    if n_pad != n:
        xt = jnp.pad(xt, ((0, 0), (0, n_pad - n)))

    total = fe * v
    kernel_fn = functools.partial(_encode_kernel, num_features=f, vocab=v)

    cost = pl.CostEstimate(
        flops=2 * n_pad * total * h,
        transcendentals=0,
        bytes_accessed=4 * n_pad * f + 4 * n_pad * h + 2 * total * h,
    )

    out = pl.pallas_call(
        kernel_fn,
        out_shape=jax.ShapeDtypeStruct((n_pad, h), jnp.float32),
        grid=(n_pad // tile,),
        in_specs=[
            pl.BlockSpec((f, tile), lambda i: (0, i)),
            pl.BlockSpec((total, h), lambda i: (0, 0)),
        ],
        out_specs=pl.BlockSpec((tile, h), lambda i: (i, 0)),
        compiler_params=pltpu.CompilerParams(
            dimension_semantics=("parallel",),
        ),
        cost_estimate=cost,
    )(xt, tab2d)

    return out[:n]


# speedup vs baseline: 4.9665x; 1.4816x over previous
import functools

import jax
import jax.numpy as jnp
from jax.experimental import pallas as pl
from jax.experimental.pallas import tpu as pltpu


def _round_up(a: int, m: int) -> int:
    return (a + m - 1) // m * m


def _encode_kernel(xt_ref, tab_ref, out_ref, *, num_features, vocab):
    tile_n = xt_ref.shape[1]
    total = num_features * vocab

    xt = xt_ref[...]
    selT = jnp.concatenate(
        [
            jnp.broadcast_to(xt[i : i + 1, :], (vocab, tile_n))
            for i in range(num_features)
        ],
        axis=0,
    )

    subT = jax.lax.broadcasted_iota(jnp.int32, (total, tile_n), 0) % vocab

    hotT = (selT == subT).astype(jnp.bfloat16)
    out_ref[...] = jax.lax.dot_general(
        hotT,
        tab_ref[...],
        (((0,), (0,)), ((), ())),
        preferred_element_type=jnp.float32,
    )


def kernel(x, tables):
    if x.ndim == 1:
        x = x[:, None]
    n, f = x.shape
    fe, v, h = tables.shape
    assert f == fe, "number of index columns must match number of tables"

    tab2d = tables.reshape(fe * v, h).astype(jnp.bfloat16)
    xt = x.astype(jnp.int32).T

    tile = max(128, min(4096, _round_up(n, 128)))
    n_pad = _round_up(n, tile)
    if n_pad != n:
        xt = jnp.pad(xt, ((0, 0), (0, n_pad - n)))

    total = fe * v
    kernel_fn = functools.partial(_encode_kernel, num_features=f, vocab=v)

    cost = pl.CostEstimate(
        flops=2 * n_pad * total * h,
        transcendentals=0,
        bytes_accessed=4 * n_pad * f + 4 * n_pad * h + 2 * total * h,
    )

    out = pl.pallas_call(
        kernel_fn,
        out_shape=jax.ShapeDtypeStruct((n_pad, h), jnp.float32),
        grid=(n_pad // tile,),
        in_specs=[
            pl.BlockSpec((f, tile), lambda i: (0, i)),
            pl.BlockSpec((total, h), lambda i: (0, 0)),
        ],
        out_specs=pl.BlockSpec((tile, h), lambda i: (i, 0)),
        compiler_params=pltpu.CompilerParams(
            dimension_semantics=("parallel",),
        ),
        cost_estimate=cost,
    )(xt, tab2d)

    return out[:n]


# speedup vs baseline: 6.6487x; 1.3387x over previous
import functools

import jax
import jax.numpy as jnp
from jax.experimental import pallas as pl
from jax.experimental.pallas import tpu as pltpu


def _round_up(a: int, m: int) -> int:
    return (a + m - 1) // m * m


def _encode_kernel(xt_ref, tab_ref, out_ref, *, num_features, vocab):
    tile_n = xt_ref.shape[1]
    total = num_features * vocab

    xt = xt_ref[...]
    selT = jnp.concatenate(
        [
            jnp.broadcast_to(xt[i : i + 1, :], (vocab, tile_n))
            for i in range(num_features)
        ],
        axis=0,
    )

    subT = jax.lax.broadcasted_iota(jnp.int32, (total, tile_n), 0) % vocab

    hotT = (selT == subT).astype(jnp.bfloat16)
    out_ref[...] = jax.lax.dot_general(
        hotT,
        tab_ref[...],
        (((0,), (0,)), ((), ())),
        preferred_element_type=jnp.float32,
    )


def kernel(x, tables):
    if x.ndim == 1:
        x = x[:, None]
    n, f = x.shape
    fe, v, h = tables.shape
    assert f == fe, "number of index columns must match number of tables"

    tab2d = tables.reshape(fe * v, h).astype(jnp.bfloat16)
    xt = x.astype(jnp.int32).T

    tile = max(128, min(8192, _round_up(n, 128)))
    n_pad = _round_up(n, tile)
    if n_pad != n:
        xt = jnp.pad(xt, ((0, 0), (0, n_pad - n)))

    total = fe * v
    kernel_fn = functools.partial(_encode_kernel, num_features=f, vocab=v)

    cost = pl.CostEstimate(
        flops=2 * n_pad * total * h,
        transcendentals=0,
        bytes_accessed=4 * n_pad * f + 4 * n_pad * h + 2 * total * h,
    )

    out = pl.pallas_call(
        kernel_fn,
        out_shape=jax.ShapeDtypeStruct((n_pad, h), jnp.float32),
        grid=(n_pad // tile,),
        in_specs=[
            pl.BlockSpec((f, tile), lambda i: (0, i)),
            pl.BlockSpec((total, h), lambda i: (0, 0)),
        ],
        out_specs=pl.BlockSpec((tile, h), lambda i: (i, 0)),
        compiler_params=pltpu.CompilerParams(
            dimension_semantics=("parallel",),
        ),
        cost_estimate=cost,
    )(xt, tab2d)

    return out[:n]


# speedup vs baseline: 7.7271x; 1.1622x over previous
import functools

import jax
import jax.numpy as jnp
from jax.experimental import pallas as pl
from jax.experimental.pallas import tpu as pltpu


def _round_up(a: int, m: int) -> int:
    return (a + m - 1) // m * m


def _encode_kernel(xt_ref, tab_ref, out_ref, *, num_features, vocab):
    tile_n = xt_ref.shape[1]
    total = num_features * vocab

    xt = xt_ref[...]
    selT = jnp.concatenate(
        [
            jnp.broadcast_to(xt[i : i + 1, :], (vocab, tile_n))
            for i in range(num_features)
        ],
        axis=0,
    )

    subT = jax.lax.broadcasted_iota(jnp.int32, (total, tile_n), 0) % vocab

    hotT = (selT == subT).astype(jnp.bfloat16)
    out_ref[...] = jax.lax.dot_general(
        hotT,
        tab_ref[...],
        (((0,), (0,)), ((), ())),
        preferred_element_type=jnp.float32,
    )


def kernel(x, tables):
    if x.ndim == 1:
        x = x[:, None]
    n, f = x.shape
    fe, v, h = tables.shape
    assert f == fe, "number of index columns must match number of tables"

    tab2d = tables.reshape(fe * v, h).astype(jnp.bfloat16)
    xt = x.astype(jnp.int32).T

    tile = max(128, min(16384, _round_up(n, 128)))
    n_pad = _round_up(n, tile)
    if n_pad != n:
        xt = jnp.pad(xt, ((0, 0), (0, n_pad - n)))

    total = fe * v
    kernel_fn = functools.partial(_encode_kernel, num_features=f, vocab=v)

    cost = pl.CostEstimate(
        flops=2 * n_pad * total * h,
        transcendentals=0,
        bytes_accessed=4 * n_pad * f + 4 * n_pad * h + 2 * total * h,
    )

    out = pl.pallas_call(
        kernel_fn,
        out_shape=jax.ShapeDtypeStruct((n_pad, h), jnp.float32),
        grid=(n_pad // tile,),
        in_specs=[
            pl.BlockSpec((f, tile), lambda i: (0, i)),
            pl.BlockSpec((total, h), lambda i: (0, 0)),
        ],
        out_specs=pl.BlockSpec((tile, h), lambda i: (i, 0)),
        compiler_params=pltpu.CompilerParams(
            dimension_semantics=("parallel",),
        ),
        cost_estimate=cost,
    )(xt, tab2d)

    return out[:n]


# speedup vs baseline: 8.0954x; 1.0477x over previous
import functools

import jax
import jax.numpy as jnp
from jax.experimental import pallas as pl
from jax.experimental.pallas import tpu as pltpu


def _round_up(a: int, m: int) -> int:
    return (a + m - 1) // m * m


def _encode_kernel(xt_ref, tab_ref, out_ref, *, num_features, vocab):
    tile_n = xt_ref.shape[1]
    total = num_features * vocab

    xt = xt_ref[...]
    selT = jnp.concatenate(
        [
            jnp.broadcast_to(xt[i : i + 1, :], (vocab, tile_n))
            for i in range(num_features)
        ],
        axis=0,
    )

    subT = jax.lax.broadcasted_iota(jnp.int32, (total, tile_n), 0) % vocab

    hotT = (selT == subT).astype(jnp.bfloat16)
    out_ref[...] = jax.lax.dot_general(
        hotT,
        tab_ref[...],
        (((0,), (0,)), ((), ())),
        preferred_element_type=jnp.float32,
    )


def kernel(x, tables):
    if x.ndim == 1:
        x = x[:, None]
    n, f = x.shape
    fe, v, h = tables.shape
    assert f == fe, "number of index columns must match number of tables"

    tab2d = tables.reshape(fe * v, h).astype(jnp.bfloat16)
    xt = x.astype(jnp.int32).T

    tile = max(128, min(32768, _round_up(n, 128)))
    n_pad = _round_up(n, tile)
    if n_pad != n:
        xt = jnp.pad(xt, ((0, 0), (0, n_pad - n)))

    total = fe * v
    kernel_fn = functools.partial(_encode_kernel, num_features=f, vocab=v)

    cost = pl.CostEstimate(
        flops=2 * n_pad * total * h,
        transcendentals=0,
        bytes_accessed=4 * n_pad * f + 4 * n_pad * h + 2 * total * h,
    )

    out = pl.pallas_call(
        kernel_fn,
        out_shape=jax.ShapeDtypeStruct((n_pad, h), jnp.float32),
        grid=(n_pad // tile,),
        in_specs=[
            pl.BlockSpec((f, tile), lambda i: (0, i)),
            pl.BlockSpec((total, h), lambda i: (0, 0)),
        ],
        out_specs=pl.BlockSpec((tile, h), lambda i: (i, 0)),
        compiler_params=pltpu.CompilerParams(
            dimension_semantics=("parallel",),
        ),
        cost_estimate=cost,
    )(xt, tab2d)

    return out[:n]


# speedup vs baseline: 8.1849x; 1.0111x over previous
import functools

import jax
import jax.numpy as jnp
from jax.experimental import pallas as pl
from jax.experimental.pallas import tpu as pltpu


def _round_up(a: int, m: int) -> int:
    return (a + m - 1) // m * m


def _encode_kernel(xt_ref, tab_ref, out_ref, *, num_features, vocab):
    tile_n = xt_ref.shape[1]
    total = num_features * vocab

    xt = xt_ref[...]
    selT = jnp.concatenate(
        [
            jnp.broadcast_to(xt[i : i + 1, :], (vocab, tile_n))
            for i in range(num_features)
        ],
        axis=0,
    )

    subT = jax.lax.broadcasted_iota(jnp.int32, (total, tile_n), 0) % vocab

    hotT = (selT == subT).astype(jnp.bfloat16)
    out_ref[...] = jax.lax.dot_general(
        hotT,
        tab_ref[...].astype(jnp.bfloat16),
        (((0,), (0,)), ((), ())),
        preferred_element_type=jnp.float32,
    )


def kernel(x, tables):
    if x.ndim == 1:
        x = x[:, None]
    n, f = x.shape
    fe, v, h = tables.shape
    assert f == fe, "number of index columns must match number of tables"

    tab2d = tables.reshape(fe * v, h)
    xt = x.astype(jnp.int32).T

    tile = max(128, min(32768, _round_up(n, 128)))
    n_pad = _round_up(n, tile)
    if n_pad != n:
        xt = jnp.pad(xt, ((0, 0), (0, n_pad - n)))

    total = fe * v
    kernel_fn = functools.partial(_encode_kernel, num_features=f, vocab=v)

    cost = pl.CostEstimate(
        flops=2 * n_pad * total * h,
        transcendentals=0,
        bytes_accessed=4 * n_pad * f + 4 * n_pad * h + 4 * total * h,
    )

    out = pl.pallas_call(
        kernel_fn,
        out_shape=jax.ShapeDtypeStruct((n_pad, h), jnp.float32),
        grid=(n_pad // tile,),
        in_specs=[
            pl.BlockSpec((f, tile), lambda i: (0, i)),
            pl.BlockSpec((total, h), lambda i: (0, 0)),
        ],
        out_specs=pl.BlockSpec((tile, h), lambda i: (i, 0)),
        compiler_params=pltpu.CompilerParams(
            dimension_semantics=("parallel",),
        ),
        cost_estimate=cost,
    )(xt, tab2d)

    return out[:n]
